# baseline (speedup 1.0000x reference)
import functools

import jax
import jax.numpy as jnp
from jax import lax
from jax.experimental import pallas as pl
from jax.experimental.pallas import tpu as pltpu
from jax.experimental.pallas import tpu_sc as plsc


def _sc_copy(pe2d):
    L, D = pe2d.shape
    info = plsc.get_sparse_core_info()
    nw = info.num_cores * info.num_subcores
    rows_per_w = L // nw

    mesh = plsc.VectorSubcoreMesh(core_axis_name="c", subcore_axis_name="s")

    @functools.partial(
        pl.kernel,
        out_type=jax.ShapeDtypeStruct((L, D), pe2d.dtype),
        mesh=mesh,
        scratch_types=[pltpu.VMEM((rows_per_w, D), pe2d.dtype)],
    )
    def copy_kernel(pe_hbm, out_hbm, buf):
        wid = lax.axis_index("s") * info.num_cores + lax.axis_index("c")
        base = wid * rows_per_w
        pltpu.sync_copy(pe_hbm.at[pl.ds(base, rows_per_w)], buf)
        pltpu.sync_copy(buf, out_hbm.at[pl.ds(base, rows_per_w)])

    return copy_kernel(pe2d)


def kernel(x, pe):
    L = x.shape[1]
    pe2d = pe.reshape(pe.shape[1], pe.shape[2])[:L]
    return _sc_copy(pe2d)[None]
